# Initial kernel scaffold; baseline (speedup 1.0000x reference)
import jax
import jax.numpy as jnp
from jax.experimental import pallas as pl


def kernel(x, edge_index, Wl1, bl1, Wr1, Wl2, bl2, Wr2, Wl3, bl3, Wr3, g1, be1, rm1, rv1, g2, be2, rm2, rv2, g3, be3, rm3, rv3, Wf1, bf1, Wf2, bf2, Wf3, bf3):
    raise NotImplementedError("write your pallas kernel here")


# speedup vs baseline: 7.1419x; 7.1419x over previous
import functools

import jax
import jax.numpy as jnp
from jax import lax
from jax.experimental import pallas as pl
from jax.experimental.pallas import tpu as pltpu
from jax.experimental.pallas import tpu_sc as plsc

EPS = 1e-5

NC = 2
NS = 16
K = 128


def _mesh():
    return plsc.VectorSubcoreMesh(core_axis_name="c", subcore_axis_name="s",
                                  num_cores=NC, num_subcores=NS)


@functools.partial(jax.jit, static_argnames=("n_acc", "ch2"))
def _sc_segment_sum(h2, src4, dst3, zrows, *, n_acc, ch2):
    HD = h2.shape[1]
    rpt = n_acc // NS

    out_type = jax.ShapeDtypeStruct((NC, n_acc, HD), jnp.float32)
    scratch = [
        pltpu.VMEM((ch2, K), jnp.int32),
        pltpu.VMEM((ch2, K), jnp.int32),
        pltpu.VMEM((K, HD), jnp.float32),
        pltpu.VMEM((K, HD), jnp.float32),
        pltpu.VMEM_SHARED((n_acc, HD), jnp.float32),
        pltpu.SemaphoreType.DMA,
        pltpu.SemaphoreType.DMA,
    ]

    def body(h_hbm, src_hbm, dst_hbm, z_hbm, p_hbm,
             src_v, dst_v, buf0, buf1, acc, sem0, sem1):
        c = lax.axis_index("c")
        s = lax.axis_index("s")

        pltpu.sync_copy(src_hbm.at[c, s], src_v)
        pltpu.sync_copy(dst_hbm.at[s], dst_v)
        r0 = s * rpt
        nfull, rem = rpt // K, rpt % K
        for r in range(nfull):
            pltpu.sync_copy(z_hbm, acc.at[pl.ds(r0 + r * K, K)])
        if rem:
            pltpu.sync_copy(z_hbm.at[pl.ds(0, rem)],
                            acc.at[pl.ds(r0 + nfull * K, rem)])
        plsc.subcore_barrier()

        pltpu.async_copy(h_hbm.at[src_v.at[0]], buf0, sem0)

        @pl.loop(0, ch2 // 2)
        def _(p):
            c0 = p * 2
            pltpu.make_async_copy(h_hbm.at[src_v.at[c0]], buf0, sem0).wait()
            pltpu.async_copy(h_hbm.at[src_v.at[c0 + 1]], buf1, sem1)
            pltpu.sync_copy(buf0, acc.at[dst_v.at[c0]], add=True)
            pltpu.make_async_copy(h_hbm.at[src_v.at[c0 + 1]], buf1, sem1).wait()

            @pl.when(c0 + 2 < ch2)
            def _():
                pltpu.async_copy(h_hbm.at[src_v.at[c0 + 2]], buf0, sem0)

            pltpu.sync_copy(buf1, acc.at[dst_v.at[c0 + 1]], add=True)

        plsc.subcore_barrier()
        pltpu.sync_copy(acc.at[pl.ds(r0, rpt)], p_hbm.at[c, pl.ds(r0, rpt)])

    k = pl.kernel(body, out_type=out_type, mesh=_mesh(),
                  scratch_types=scratch,
                  compiler_params=pltpu.CompilerParams(
                      use_tc_tiling_on_sc=False))
    return k(h2, src4, dst3, zrows)


@functools.partial(jax.jit, static_argnames=("n_acc", "ch"))
def _sc_degree(dst3, zcnt, ones, *, n_acc, ch):
    rpt = n_acc // NS

    out_type = jax.ShapeDtypeStruct((NC, n_acc, 16), jnp.float32)
    scratch = [
        pltpu.VMEM((ch, K), jnp.int32),
        pltpu.VMEM((K, 16), jnp.float32),
        pltpu.VMEM_SHARED((n_acc, 16), jnp.float32),
    ]

    def body(dst_hbm, zc_hbm, ones_hbm, c_hbm, dst_v, ones_v, cnt):
        c = lax.axis_index("c")
        s = lax.axis_index("s")
        w = c * NS + s

        pltpu.sync_copy(dst_hbm.at[w], dst_v)
        pltpu.sync_copy(ones_hbm, ones_v)
        r0 = s * rpt
        pltpu.sync_copy(zc_hbm, cnt.at[pl.ds(r0, rpt)])
        plsc.subcore_barrier()

        @pl.loop(0, ch)
        def _(j):
            pltpu.sync_copy(ones_v, cnt.at[dst_v.at[j]], add=True)

        plsc.subcore_barrier()
        pltpu.sync_copy(cnt.at[pl.ds(r0, rpt)], c_hbm.at[c, pl.ds(r0, rpt)])

    k = pl.kernel(body, out_type=out_type, mesh=_mesh(),
                  scratch_types=scratch,
                  compiler_params=pltpu.CompilerParams(
                      use_tc_tiling_on_sc=False))
    return k(dst3, zcnt, ones)


def _tc_layer(P, C, h, WlT, bl, WrT, g, be, rm, rv, *, residual, bm):
    N, D = h.shape

    def body(p_ref, c_ref, h_ref, wl_ref, bl_ref, wr_ref, g_ref, be_ref,
             rm_ref, rv_ref, o_ref):
        pr = jnp.concatenate([p_ref[0], p_ref[1]], axis=1)
        cnt = c_ref[0] + c_ref[1]
        inv = 1.0 / jnp.maximum(cnt[:, 0:1], 1.0)
        agg = pr * inv
        hblk = h_ref[...]
        z = jnp.dot(agg, wl_ref[...], preferred_element_type=jnp.float32)
        z = z + jnp.dot(hblk, wr_ref[...], preferred_element_type=jnp.float32)
        z = z + bl_ref[...]
        scale = g_ref[...] * lax.rsqrt(rv_ref[...] + EPS)
        z = (z - rm_ref[...]) * scale + be_ref[...]
        z = jnp.maximum(z, 0.0)
        if residual:
            z = z + hblk
        o_ref[...] = z

    grid = (N // bm,)
    return pl.pallas_call(
        body,
        grid=grid,
        in_specs=[
            pl.BlockSpec((2, bm, D // 2), lambda i: (0, i, 0)),
            pl.BlockSpec((2, bm, 16), lambda i: (0, i, 0)),
            pl.BlockSpec((bm, D), lambda i: (i, 0)),
            pl.BlockSpec((D, D), lambda i: (0, 0)),
            pl.BlockSpec((1, D), lambda i: (0, 0)),
            pl.BlockSpec((D, D), lambda i: (0, 0)),
            pl.BlockSpec((1, D), lambda i: (0, 0)),
            pl.BlockSpec((1, D), lambda i: (0, 0)),
            pl.BlockSpec((1, D), lambda i: (0, 0)),
            pl.BlockSpec((1, D), lambda i: (0, 0)),
        ],
        out_specs=pl.BlockSpec((bm, D), lambda i: (i, 0)),
        out_shape=jax.ShapeDtypeStruct((N, D), jnp.float32),
    )(P[:, :N], C[:, :N], h, WlT, bl, WrT, g, be, rm, rv)


def _tc_head(h, W1T, b1, W2T, b2, W3T, b3, *, bm):
    N, D = h.shape
    H1 = W1T.shape[1]
    H2 = W2T.shape[1]

    def body(h_ref, w1_ref, b1_ref, w2_ref, b2_ref, w3_ref, b3_ref, o_ref):
        z = jnp.dot(h_ref[...], w1_ref[...], preferred_element_type=jnp.float32)
        z = jnp.maximum(z + b1_ref[...], 0.0)
        z = jnp.dot(z, w2_ref[...], preferred_element_type=jnp.float32)
        z = jnp.maximum(z + b2_ref[...], 0.0)
        z = jnp.dot(z, w3_ref[...], preferred_element_type=jnp.float32)
        z = z + b3_ref[...]
        o_ref[...] = jax.nn.sigmoid(z)

    return pl.pallas_call(
        body,
        grid=(N // bm,),
        in_specs=[
            pl.BlockSpec((bm, D), lambda i: (i, 0)),
            pl.BlockSpec((D, H1), lambda i: (0, 0)),
            pl.BlockSpec((1, H1), lambda i: (0, 0)),
            pl.BlockSpec((H1, H2), lambda i: (0, 0)),
            pl.BlockSpec((1, H2), lambda i: (0, 0)),
            pl.BlockSpec((H2, 1), lambda i: (0, 0)),
            pl.BlockSpec((1, 1), lambda i: (0, 0)),
        ],
        out_specs=pl.BlockSpec((bm, 1), lambda i: (i, 0)),
        out_shape=jax.ShapeDtypeStruct((N, 1), jnp.float32),
    )(h, W1T, b1, W2T, b2, W3T, b3)


def kernel(x, edge_index, Wl1, bl1, Wr1, Wl2, bl2, Wr2, Wl3, bl3, Wr3,
           g1, be1, rm1, rv1, g2, be2, rm2, rv2, g3, be3, rm3, rv3,
           Wf1, bf1, Wf2, bf2, Wf3, bf3):
    N, D = x.shape
    E = edge_index.shape[1]
    NW = NC * NS

    ch = -(-E // (NW * K))
    ch = ch + (ch % 2)
    ch2 = 2 * ch
    e_pad = NS * ch2 * K
    n_acc = -(-(N + 32) // 128) * 128
    n_dummy = n_acc - N
    rpt = n_acc // NS

    pad = e_pad - E
    src = edge_index[0].astype(jnp.int32)
    dst = edge_index[1].astype(jnp.int32)
    ar = jnp.arange(pad, dtype=jnp.int32)
    src_p = jnp.concatenate([src, (ar * 7) % N])
    dst_p = jnp.concatenate([dst, N + ar % n_dummy])
    s2 = (src_p * 2).reshape(ch2, NS, K).swapaxes(0, 1)
    src4 = jnp.stack([s2, s2 + 1])
    dst3 = dst_p.reshape(ch2, NS, K).swapaxes(0, 1)
    dst32 = dst_p.reshape(ch, NW, K).swapaxes(0, 1)

    zrows = jnp.zeros((K, D // 2), jnp.float32)
    zcnt = jnp.zeros((rpt, 16), jnp.float32)
    ones = jnp.ones((K, 16), jnp.float32)

    r2 = lambda v: v.reshape(1, -1)
    h2v = lambda h: h.reshape(2 * N, D // 2)
    bm = 1000

    C = _sc_degree(dst32, zcnt, ones, n_acc=n_acc, ch=ch)
    P1 = _sc_segment_sum(h2v(x), src4, dst3, zrows, n_acc=n_acc, ch2=ch2)
    h1 = _tc_layer(P1, C, x, Wl1.T, r2(bl1), Wr1.T, r2(g1), r2(be1),
                   r2(rm1), r2(rv1), residual=False, bm=bm)
    P2 = _sc_segment_sum(h2v(h1), src4, dst3, zrows, n_acc=n_acc, ch2=ch2)
    h2 = _tc_layer(P2, C, h1, Wl2.T, r2(bl2), Wr2.T, r2(g2), r2(be2),
                   r2(rm2), r2(rv2), residual=True, bm=bm)
    P3 = _sc_segment_sum(h2v(h2), src4, dst3, zrows, n_acc=n_acc, ch2=ch2)
    h3 = _tc_layer(P3, C, h2, Wl3.T, r2(bl3), Wr3.T, r2(g3), r2(be3),
                   r2(rm3), r2(rv3), residual=True, bm=bm)
    return _tc_head(h3, Wf1.T, r2(bf1), Wf2.T, r2(bf2), Wf3.T, r2(bf3),
                    bm=bm)


# speedup vs baseline: 9.1488x; 1.2810x over previous
import functools

import jax
import jax.numpy as jnp
from jax import lax
from jax.experimental import pallas as pl
from jax.experimental.pallas import tpu as pltpu
from jax.experimental.pallas import tpu_sc as plsc

EPS = 1e-5

NC = 2
NS = 16
K = 128


def _mesh():
    return plsc.VectorSubcoreMesh(core_axis_name="c", subcore_axis_name="s",
                                  num_cores=NC, num_subcores=NS)


NBUF = 4


@functools.partial(jax.jit, static_argnames=("n_acc", "ch2", "with_cnt"))
def _sc_segment_sum(h2, src4, dst3, zrows, zcnt, ones, *, n_acc, ch2,
                    with_cnt):
    HD = h2.shape[1]
    rpt = n_acc // NS

    out_type = [jax.ShapeDtypeStruct((NC, n_acc, HD), jnp.float32)]
    scratch = [
        pltpu.VMEM((ch2, K), jnp.int32),
        pltpu.VMEM((ch2, K), jnp.int32),
        [pltpu.VMEM((K, HD), jnp.float32) for _ in range(NBUF)],
        pltpu.VMEM_SHARED((n_acc, HD), jnp.float32),
        [pltpu.SemaphoreType.DMA for _ in range(NBUF)],
        [pltpu.SemaphoreType.DMA for _ in range(NBUF)],
    ]
    if with_cnt:
        out_type.append(jax.ShapeDtypeStruct((NC, n_acc, 16), jnp.float32))
        scratch.append(pltpu.VMEM((K, 16), jnp.float32))
        scratch.append(pltpu.VMEM_SHARED((n_acc, 16), jnp.float32))

    def body(h_hbm, src_hbm, dst_hbm, z_hbm, zc_hbm, ones_hbm, *rest):
        if with_cnt:
            (p_hbm, c_hbm, src_v, dst_v, bufs, acc, gsem, ssem,
             ones_v, cnt) = rest
        else:
            (p_hbm, src_v, dst_v, bufs, acc, gsem, ssem) = rest
        c = lax.axis_index("c")
        s = lax.axis_index("s")

        pltpu.sync_copy(src_hbm.at[c, s], src_v)
        pltpu.sync_copy(dst_hbm.at[s], dst_v)
        r0 = s * rpt
        nfull, rem = rpt // K, rpt % K
        for r in range(nfull):
            pltpu.sync_copy(z_hbm, acc.at[pl.ds(r0 + r * K, K)])
        if rem:
            pltpu.sync_copy(z_hbm.at[pl.ds(0, rem)],
                            acc.at[pl.ds(r0 + nfull * K, rem)])
        if with_cnt:
            pltpu.sync_copy(zc_hbm, cnt.at[pl.ds(r0, rpt)])
            pltpu.sync_copy(ones_hbm, ones_v)
        plsc.subcore_barrier()

        def gather(cc, b):
            return pltpu.async_copy(h_hbm.at[src_v.at[cc]], bufs[b], gsem[b])

        def scatter(cc, b):
            return pltpu.async_copy(bufs[b], acc.at[dst_v.at[cc]], ssem[b],
                                    add=True)

        for b in range(NBUF):
            gather(b, b)

        @pl.loop(0, ch2 // NBUF)
        def _(r):
            c0 = r * NBUF
            for b in range(NBUF):
                cc = c0 + b
                pltpu.make_async_copy(h_hbm.at[src_v.at[cc]], bufs[b],
                                      gsem[b]).wait()
                scatter(cc, b)
                if with_cnt:
                    @pl.when(c == (b % 2))
                    def _():
                        pltpu.sync_copy(ones_v, cnt.at[dst_v.at[cc]],
                                        add=True)
            for b in range(NBUF):
                cc = c0 + b + NBUF

                @pl.when(cc < ch2)
                def _():
                    pltpu.make_async_copy(bufs[b], acc.at[dst_v.at[c0 + b]],
                                          ssem[b]).wait()
                    gather(cc, b)

        for b in range(NBUF):
            pltpu.make_async_copy(bufs[b], acc.at[dst_v.at[ch2 - NBUF + b]],
                                  ssem[b]).wait()

        plsc.subcore_barrier()
        pltpu.sync_copy(acc.at[pl.ds(r0, rpt)], p_hbm.at[c, pl.ds(r0, rpt)])
        if with_cnt:
            pltpu.sync_copy(cnt.at[pl.ds(r0, rpt)], c_hbm.at[c, pl.ds(r0, rpt)])

    k = pl.kernel(body, out_type=out_type, mesh=_mesh(),
                  scratch_types=scratch,
                  compiler_params=pltpu.CompilerParams(
                      use_tc_tiling_on_sc=False))
    return k(h2, src4, dst3, zrows, zcnt, ones)


def _tc_layer(P, C, h, WlT, bl, WrT, g, be, rm, rv, *, residual, bm,
              head=None):
    N, D = h.shape

    def body(p_ref, c_ref, h_ref, wl_ref, bl_ref, wr_ref, g_ref, be_ref,
             rm_ref, rv_ref, *rest):
        o_ref = rest[-1]
        pr = jnp.concatenate([p_ref[0], p_ref[1]], axis=1)
        cnt = c_ref[0] + c_ref[1]
        inv = 1.0 / jnp.maximum(cnt[:, 0:1], 1.0)
        agg = pr * inv
        hblk = h_ref[...]
        z = jnp.dot(agg, wl_ref[...], preferred_element_type=jnp.float32)
        z = z + jnp.dot(hblk, wr_ref[...], preferred_element_type=jnp.float32)
        z = z + bl_ref[...]
        scale = g_ref[...] * lax.rsqrt(rv_ref[...] + EPS)
        z = (z - rm_ref[...]) * scale + be_ref[...]
        z = jnp.maximum(z, 0.0)
        if residual:
            z = z + hblk
        if head is not None:
            w1_ref, b1_ref, w2_ref, b2_ref, w3_ref, b3_ref = rest[:-1]
            z = jnp.dot(z, w1_ref[...], preferred_element_type=jnp.float32)
            z = jnp.maximum(z + b1_ref[...], 0.0)
            z = jnp.dot(z, w2_ref[...], preferred_element_type=jnp.float32)
            z = jnp.maximum(z + b2_ref[...], 0.0)
            z = jnp.dot(z, w3_ref[...], preferred_element_type=jnp.float32)
            z = jax.nn.sigmoid(z + b3_ref[...])
        o_ref[...] = z

    in_specs = [
        pl.BlockSpec((2, bm, D // 2), lambda i: (0, i, 0)),
        pl.BlockSpec((2, bm, 16), lambda i: (0, i, 0)),
        pl.BlockSpec((bm, D), lambda i: (i, 0)),
        pl.BlockSpec((D, D), lambda i: (0, 0)),
        pl.BlockSpec((1, D), lambda i: (0, 0)),
        pl.BlockSpec((D, D), lambda i: (0, 0)),
        pl.BlockSpec((1, D), lambda i: (0, 0)),
        pl.BlockSpec((1, D), lambda i: (0, 0)),
        pl.BlockSpec((1, D), lambda i: (0, 0)),
        pl.BlockSpec((1, D), lambda i: (0, 0)),
    ]
    args = [P[:, :N], C[:, :N], h, WlT, bl, WrT, g, be, rm, rv]
    if head is None:
        out_w = D
    else:
        W1T, b1, W2T, b2, W3T, b3 = head
        H1, H2 = W1T.shape[1], W2T.shape[1]
        in_specs += [
            pl.BlockSpec((D, H1), lambda i: (0, 0)),
            pl.BlockSpec((1, H1), lambda i: (0, 0)),
            pl.BlockSpec((H1, H2), lambda i: (0, 0)),
            pl.BlockSpec((1, H2), lambda i: (0, 0)),
            pl.BlockSpec((H2, 1), lambda i: (0, 0)),
            pl.BlockSpec((1, 1), lambda i: (0, 0)),
        ]
        args += [W1T, b1, W2T, b2, W3T, b3]
        out_w = 1

    return pl.pallas_call(
        body,
        grid=(N // bm,),
        in_specs=in_specs,
        out_specs=pl.BlockSpec((bm, out_w), lambda i: (i, 0)),
        out_shape=jax.ShapeDtypeStruct((N, out_w), jnp.float32),
    )(*args)


def kernel(x, edge_index, Wl1, bl1, Wr1, Wl2, bl2, Wr2, Wl3, bl3, Wr3,
           g1, be1, rm1, rv1, g2, be2, rm2, rv2, g3, be3, rm3, rv3,
           Wf1, bf1, Wf2, bf2, Wf3, bf3):
    N, D = x.shape
    E = edge_index.shape[1]
    NW = NC * NS

    ch = -(-E // (NW * K))
    ch = ch + (ch % 2)
    ch2 = 2 * ch
    e_pad = NS * ch2 * K
    n_acc = -(-(N + 32) // 128) * 128
    n_dummy = n_acc - N
    rpt = n_acc // NS

    pad = e_pad - E
    src = edge_index[0].astype(jnp.int32)
    dst = edge_index[1].astype(jnp.int32)
    ar = jnp.arange(pad, dtype=jnp.int32)
    src_p = jnp.concatenate([src, (ar * 7) % N])
    dst_p = jnp.concatenate([dst, N + ar % n_dummy])
    s2 = (src_p * 2).reshape(ch2, NS, K).swapaxes(0, 1)
    src4 = jnp.stack([s2, s2 + 1])
    dst3 = dst_p.reshape(ch2, NS, K).swapaxes(0, 1)

    zrows = jnp.zeros((K, D // 2), jnp.float32)
    zcnt = jnp.zeros((rpt, 16), jnp.float32)
    ones = jnp.ones((K, 16), jnp.float32)

    r2 = lambda v: v.reshape(1, -1)
    h2v = lambda h: h.reshape(2 * N, D // 2)
    bm = 1000

    P1, C = _sc_segment_sum(h2v(x), src4, dst3, zrows, zcnt, ones,
                            n_acc=n_acc, ch2=ch2, with_cnt=True)
    h1 = _tc_layer(P1, C, x, Wl1.T, r2(bl1), Wr1.T, r2(g1), r2(be1),
                   r2(rm1), r2(rv1), residual=False, bm=bm)
    (P2,) = _sc_segment_sum(h2v(h1), src4, dst3, zrows, zcnt, ones,
                            n_acc=n_acc, ch2=ch2, with_cnt=False)
    h2 = _tc_layer(P2, C, h1, Wl2.T, r2(bl2), Wr2.T, r2(g2), r2(be2),
                   r2(rm2), r2(rv2), residual=True, bm=bm)
    (P3,) = _sc_segment_sum(h2v(h2), src4, dst3, zrows, zcnt, ones,
                            n_acc=n_acc, ch2=ch2, with_cnt=False)
    return _tc_layer(P3, C, h2, Wl3.T, r2(bl3), Wr3.T, r2(g3), r2(be3),
                     r2(rm3), r2(rv3), residual=True, bm=bm,
                     head=(Wf1.T, r2(bf1), Wf2.T, r2(bf2), Wf3.T, r2(bf3)))
